# Initial kernel scaffold; baseline (speedup 1.0000x reference)
import functools

import jax
import jax.numpy as jnp
from jax import lax
from jax.experimental import pallas as pl
from jax.experimental.pallas import tpu as pltpu
from jax.experimental.pallas import tpu_sc as plsc

N = 10000
E = 320000
D = 128
H = 8
OUT = 128
DW = H * OUT
EPS = 1e-5

NC = 2
NS = 16
NW = NC * NS
EPW = E // NW
B = 80
CHUNKS = EPW // B
ROWS_PER_SID = N // NS

NBLK = 2000
GRID = N // NBLK


def _edge_body(with_cnt, *refs):
    if with_cnt:
        (src_hbm, dst_hbm, yut_hbm, y_hbm, z128_hbm, z16_hbm,
         agg_out, cnt_out,
         src_v, dst_v, yus_v, yud_v, y_v, coef_v, msg_v, wrow_v,
         agg_sp, cnt_sp, sem0, sem1, sem2) = refs
    else:
        (src_hbm, dst_hbm, yut_hbm, y_hbm, z128_hbm,
         agg_out,
         src_v, dst_v, yus_v, yud_v, y_v, coef_v, msg_v,
         agg_sp, sem0, sem1, sem2) = refs

    cid = lax.axis_index("c")
    sid = lax.axis_index("s")
    wid = sid * NC + cid

    row0 = sid * ROWS_PER_SID
    pltpu.sync_copy(z128_hbm, agg_sp.at[pl.ds(row0, ROWS_PER_SID)])
    if with_cnt:
        pltpu.sync_copy(z16_hbm, cnt_sp.at[pl.ds(row0, ROWS_PER_SID)])
        pltpu.sync_copy(z16_hbm.at[pl.ds(0, B)], wrow_v)
    plsc.subcore_barrier()

    base0 = wid * EPW
    iota = lax.iota(jnp.int32, 16)

    def chunk(i, carry):
        base = pl.multiple_of(base0 + i * B, 8)
        pltpu.sync_copy(src_hbm.at[pl.ds(base, B)], src_v)
        pltpu.sync_copy(dst_hbm.at[pl.ds(base, B)], dst_v)
        cp0 = pltpu.async_copy(yut_hbm.at[src_v], yus_v, sem0)
        cp1 = pltpu.async_copy(yut_hbm.at[dst_v], yud_v, sem1)
        cp2 = pltpu.async_copy(y_hbm.at[src_v], y_v, sem2)
        cp0.wait()
        cp1.wait()

        for g in range(B // 16):
            rows = g * 16 + iota
            sg = src_v[pl.ds(g * 16, 16)]
            dg = dst_v[pl.ds(g * 16, 16)]
            w = jnp.where(sg != dg, 1.0, 0.0).astype(jnp.float32)
            dif = []
            for h in range(H):
                a = plsc.load_gather(yud_v, [rows, jnp.full((16,), H + h, jnp.int32)])
                b = plsc.load_gather(yus_v, [rows, jnp.full((16,), h, jnp.int32)])
                dif.append(a - b)
            m = dif[0]
            for h in range(1, H):
                m = jnp.maximum(m, dif[h])
            ex = [jnp.exp(dif[h] - m) for h in range(H)]
            tot = ex[0]
            for h in range(1, H):
                tot = tot + ex[h]
            scale = w / tot
            for h in range(H):
                plsc.store_scatter(
                    coef_v, [rows, jnp.full((16,), h, jnp.int32)], ex[h] * scale)
            if with_cnt:
                plsc.store_scatter(
                    wrow_v, [rows, jnp.full((16,), 0, jnp.int32)], w)

        cp2.wait()

        def edge(e, c2):
            accs = [jnp.zeros((16,), jnp.float32) for _ in range(OUT // 16)]
            for h in range(H):
                ah = plsc.load_gather(
                    coef_v,
                    [jnp.full((16,), e, jnp.int32), jnp.full((16,), h, jnp.int32)])
                for k in range(OUT // 16):
                    accs[k] = accs[k] + ah * y_v[e, pl.ds(h * OUT + k * 16, 16)]
            for k in range(OUT // 16):
                msg_v[e, pl.ds(k * 16, 16)] = accs[k]
            return c2

        lax.fori_loop(0, B, edge, 0)

        pltpu.sync_copy(msg_v, agg_sp.at[dst_v], add=True)
        if with_cnt:
            pltpu.sync_copy(wrow_v, cnt_sp.at[dst_v], add=True)
        return carry

    lax.fori_loop(0, CHUNKS, chunk, 0)

    plsc.subcore_barrier()
    pltpu.sync_copy(agg_sp.at[pl.ds(row0, ROWS_PER_SID)],
                    agg_out.at[cid, pl.ds(row0, ROWS_PER_SID)])
    if with_cnt:
        pltpu.sync_copy(cnt_sp.at[pl.ds(row0, ROWS_PER_SID)],
                        cnt_out.at[cid, pl.ds(row0, ROWS_PER_SID)])


def _make_edge_kernel(with_cnt):
    mesh = plsc.VectorSubcoreMesh(core_axis_name="c", subcore_axis_name="s")
    out_type = [jax.ShapeDtypeStruct((NC, N, OUT), jnp.float32)]
    scratch = [
        pltpu.VMEM((B,), jnp.int32),
        pltpu.VMEM((B,), jnp.int32),
        pltpu.VMEM((B, 2 * H), jnp.float32),
        pltpu.VMEM((B, 2 * H), jnp.float32),
        pltpu.VMEM((B, DW), jnp.float32),
        pltpu.VMEM((B, H), jnp.float32),
        pltpu.VMEM((B, OUT), jnp.float32),
    ]
    if with_cnt:
        out_type.append(jax.ShapeDtypeStruct((NC, N, 16), jnp.float32))
        scratch.append(pltpu.VMEM((B, 16), jnp.float32))
    scratch.append(pltpu.VMEM_SHARED((N, OUT), jnp.float32))
    if with_cnt:
        scratch.append(pltpu.VMEM_SHARED((N, 16), jnp.float32))
    scratch += [pltpu.SemaphoreType.DMA] * 3
    return pl.kernel(
        functools.partial(_edge_body, with_cnt),
        out_type=out_type,
        mesh=mesh,
        scratch_types=scratch,
    )


_edge_kernel_cnt = _make_edge_kernel(True)
_edge_kernel = _make_edge_kernel(False)


def _node_kernel(x_ref, scale_ref, shift_ref, wt_ref, ut_ref, c_ref,
                 y_ref, yut_ref, self_ref):
    xn = x_ref[...] * scale_ref[...] + shift_ref[...]
    y = jnp.dot(xn, wt_ref[...], preferred_element_type=jnp.float32)
    y_ref[...] = y
    yu = jnp.dot(xn, ut_ref[...], preferred_element_type=jnp.float32)
    yut_ref[...] = jnp.concatenate([yu, yu + c_ref[...]], axis=1)
    c = c_ref[...]
    p = jax.nn.softmax(c, axis=1)
    self_ref[...] = jnp.sum(
        y.reshape(NBLK, H, OUT) * p.reshape(1, H, 1), axis=1)


def _node_transform(x, scale, shift, wt, ut, c):
    return pl.pallas_call(
        _node_kernel,
        grid=(GRID,),
        in_specs=[
            pl.BlockSpec((NBLK, D), lambda i: (i, 0)),
            pl.BlockSpec((1, D), lambda i: (0, 0)),
            pl.BlockSpec((1, D), lambda i: (0, 0)),
            pl.BlockSpec((D, DW), lambda i: (0, 0)),
            pl.BlockSpec((D, H), lambda i: (0, 0)),
            pl.BlockSpec((1, H), lambda i: (0, 0)),
        ],
        out_specs=[
            pl.BlockSpec((NBLK, DW), lambda i: (i, 0)),
            pl.BlockSpec((NBLK, 2 * H), lambda i: (i, 0)),
            pl.BlockSpec((NBLK, OUT), lambda i: (i, 0)),
        ],
        out_shape=[
            jax.ShapeDtypeStruct((N, DW), jnp.float32),
            jax.ShapeDtypeStruct((N, 2 * H), jnp.float32),
            jax.ShapeDtypeStruct((N, OUT), jnp.float32),
        ],
    )(x, scale, shift, wt, ut, c)


def _post_kernel_mid(agg_ref, cnt_ref, self_ref, b_ref,
                     out_ref, sum_ref, sq_ref):
    a = agg_ref[0] + agg_ref[1] + self_ref[...]
    cnt = cnt_ref[0, :, 0:1] + cnt_ref[1, :, 0:1] + 1.0
    t = a / jnp.maximum(cnt, 1.0) + b_ref[...]
    t = jnp.maximum(t, 0.0)
    out_ref[...] = t
    sum_ref[...] = jnp.sum(t, axis=0, keepdims=True)
    sq_ref[...] = jnp.sum(t * t, axis=0, keepdims=True)


def _post_kernel_last(agg_ref, cnt_ref, self_ref, b_ref, out_ref):
    a = agg_ref[0] + agg_ref[1] + self_ref[...]
    cnt = cnt_ref[0, :, 0:1] + cnt_ref[1, :, 0:1] + 1.0
    out_ref[...] = a / jnp.maximum(cnt, 1.0) + b_ref[...]


def _post(aggp, cntp, selfmsg, bias, last):
    in_specs = [
        pl.BlockSpec((NC, NBLK, OUT), lambda i: (0, i, 0)),
        pl.BlockSpec((NC, NBLK, 16), lambda i: (0, i, 0)),
        pl.BlockSpec((NBLK, OUT), lambda i: (i, 0)),
        pl.BlockSpec((1, OUT), lambda i: (0, 0)),
    ]
    if last:
        return pl.pallas_call(
            _post_kernel_last,
            grid=(GRID,),
            in_specs=in_specs,
            out_specs=pl.BlockSpec((NBLK, OUT), lambda i: (i, 0)),
            out_shape=jax.ShapeDtypeStruct((N, OUT), jnp.float32),
        )(aggp, cntp, selfmsg, bias)
    return pl.pallas_call(
        _post_kernel_mid,
        grid=(GRID,),
        in_specs=in_specs,
        out_specs=[
            pl.BlockSpec((NBLK, OUT), lambda i: (i, 0)),
            pl.BlockSpec((1, OUT), lambda i: (i, 0)),
            pl.BlockSpec((1, OUT), lambda i: (i, 0)),
        ],
        out_shape=[
            jax.ShapeDtypeStruct((N, OUT), jnp.float32),
            jax.ShapeDtypeStruct((GRID, OUT), jnp.float32),
            jax.ShapeDtypeStruct((GRID, OUT), jnp.float32),
        ],
    )(aggp, cntp, selfmsg, bias)


def kernel(x, edge_index, W0, U0, C0, B0, W1, U1, C1, B1, W2, U2, C2, B2,
           G0, BT0, G1, BT1):
    src = edge_index[0]
    dst = edge_index[1]
    z128 = jnp.zeros((ROWS_PER_SID, OUT), jnp.float32)
    z16 = jnp.zeros((ROWS_PER_SID, 16), jnp.float32)

    ones = jnp.ones((1, D), jnp.float32)
    zeros = jnp.zeros((1, D), jnp.float32)

    def layer(h, scale, shift, W, U, C, bias, cntp, last):
        y, yut, selfmsg = _node_transform(
            h, scale, shift, W.T, U.T, C.reshape(1, H))
        if cntp is None:
            aggp, cntp = _edge_kernel_cnt(src, dst, yut, y, z128, z16)
        else:
            aggp = _edge_kernel(src, dst, yut, y, z128)
        res = _post(aggp, cntp, selfmsg, bias.reshape(1, OUT), last)
        return res, cntp

    (t0, s0, q0), cntp = layer(x, ones, zeros, W0, U0, C0, B0, None, False)
    mu0 = jnp.sum(s0, axis=0) / N
    var0 = jnp.sum(q0, axis=0) / N - mu0 * mu0
    sc0 = G0 / jnp.sqrt(var0 + EPS)
    sh0 = BT0 - mu0 * sc0

    (t1, s1, q1), _ = layer(t0, sc0.reshape(1, D), sh0.reshape(1, D),
                            W1, U1, C1, B1, cntp, False)
    mu1 = jnp.sum(s1, axis=0) / N
    var1 = jnp.sum(q1, axis=0) / N - mu1 * mu1
    sc1 = G1 / jnp.sqrt(var1 + EPS)
    sh1 = BT1 - mu1 * sc1

    out, _ = layer(t1, sc1.reshape(1, D), sh1.reshape(1, D),
                   W2, U2, C2, B2, cntp, True)
    return out


# speedup vs baseline: 2.3591x; 2.3591x over previous
import functools

import jax
import jax.numpy as jnp
from jax import lax
from jax.experimental import pallas as pl
from jax.experimental.pallas import tpu as pltpu
from jax.experimental.pallas import tpu_sc as plsc

N = 10000
E = 320000
D = 128
H = 8
OUT = 128
DW = H * OUT
EPS = 1e-5

NC = 2
NS = 16
NW = NC * NS
EPW = E // NW
BA = 80
CA = EPW // BA
BB = 16
CB = EPW // BB
NPAD = 10240
ROWS_PER_SID = NPAD // NS

NBLK = 2000
GRID = N // NBLK

_SC_PARAMS = pltpu.CompilerParams(needs_layout_passes=False)


def _sc_mesh():
    return plsc.VectorSubcoreMesh(core_axis_name="c", subcore_axis_name="s",
                                  num_cores=NC, num_subcores=NS)


def _coef_body(src_hbm, dst_hbm, yut_hbm, c_hbm, cf_out,
               src_v, dst_v, tab_v, cbuf_v, coef_v):
    cid = lax.axis_index("c")
    sid = lax.axis_index("s")
    wid = sid * NC + cid
    pltpu.sync_copy(yut_hbm, tab_v)
    pltpu.sync_copy(c_hbm, cbuf_v)
    iota = lax.iota(jnp.int32, 16)
    ch = [plsc.load_gather(cbuf_v, [jnp.full((16,), h, jnp.int32)])
          for h in range(H)]
    base0 = wid * EPW

    def chunk(i, carry):
        base = pl.multiple_of(base0 + i * BA, 8)
        pltpu.sync_copy(src_hbm.at[pl.ds(base, BA)], src_v)
        pltpu.sync_copy(dst_hbm.at[pl.ds(base, BA)], dst_v)
        for g in range(BA // 16):
            sg = src_v[pl.ds(g * 16, 16)]
            dg = dst_v[pl.ds(g * 16, 16)]
            w = jnp.where(sg != dg, 1.0, 0.0)
            sg8 = sg * H
            dg8 = dg * H
            dif = []
            for h in range(H):
                hv = jnp.full((16,), h, jnp.int32)
                a = plsc.load_gather(tab_v, [dg8 + hv]) + ch[h]
                b = plsc.load_gather(tab_v, [sg8 + hv])
                dif.append(a - b)
            m = dif[0]
            for h in range(1, H):
                m = jnp.maximum(m, dif[h])
            ex = [jnp.exp(dif[h] - m) for h in range(H)]
            tot = ex[0]
            for h in range(1, H):
                tot = tot + ex[h]
            scale = w / tot
            rows = (g * 16 + iota) * H
            for h in range(H):
                plsc.store_scatter(coef_v, [rows + h], ex[h] * scale)
        pltpu.sync_copy(coef_v, cf_out.at[pl.ds(base * H, BA * H)])
        return carry

    lax.fori_loop(0, CA, chunk, 0)


def _make_coef_kernel():
    return pl.kernel(
        _coef_body,
        out_type=[jax.ShapeDtypeStruct((E * H,), jnp.float32)],
        mesh=_sc_mesh(),
        compiler_params=_SC_PARAMS,
        scratch_types=[
            pltpu.VMEM((BA,), jnp.int32),
            pltpu.VMEM((BA,), jnp.int32),
            pltpu.VMEM((H * N,), jnp.float32),
            pltpu.VMEM((H,), jnp.float32),
            pltpu.VMEM((BA * H,), jnp.float32),
        ],
    )


def _agg_body(src_hbm, dst_hbm, y_hbm, cf_hbm, z_hbm, agg_out,
              src_v, dst_v, y_v, cf_v, msg_v, agg_sp, sem0):
    cid = lax.axis_index("c")
    sid = lax.axis_index("s")
    wid = sid * NC + cid
    row0 = sid * ROWS_PER_SID
    pltpu.sync_copy(z_hbm, agg_sp.at[pl.ds(row0, ROWS_PER_SID)])
    plsc.subcore_barrier()
    base0 = wid * EPW

    def chunk(i, carry):
        base = pl.multiple_of(base0 + i * BB, 8)
        pltpu.sync_copy(src_hbm.at[pl.ds(base, BB)], src_v)
        pltpu.sync_copy(dst_hbm.at[pl.ds(base, BB)], dst_v)
        pltpu.sync_copy(cf_hbm.at[pl.ds(base * H, BB * H)], cf_v)
        pltpu.async_copy(y_hbm.at[src_v], y_v, sem0).wait()

        def edge(e, c2):
            accs = [jnp.zeros((16,), jnp.float32) for _ in range(OUT // 16)]
            for h in range(H):
                ah = plsc.load_gather(
                    cf_v, [jnp.full((16,), h, jnp.int32) + e * H])
                for k in range(OUT // 16):
                    accs[k] = accs[k] + ah * y_v[e, pl.ds(h * OUT + k * 16, 16)]
            for k in range(OUT // 16):
                msg_v[e, pl.ds(k * 16, 16)] = accs[k]
            return c2

        lax.fori_loop(0, BB, edge, 0)
        pltpu.sync_copy(msg_v, agg_sp.at[dst_v], add=True)
        return carry

    lax.fori_loop(0, CB, chunk, 0)

    plsc.subcore_barrier()
    pltpu.sync_copy(agg_sp.at[pl.ds(row0, ROWS_PER_SID)],
                    agg_out.at[cid, pl.ds(row0, ROWS_PER_SID)])


def _make_agg_kernel():
    return pl.kernel(
        _agg_body,
        out_type=[jax.ShapeDtypeStruct((NC, NPAD, OUT), jnp.float32)],
        mesh=_sc_mesh(),
        compiler_params=_SC_PARAMS,
        scratch_types=[
            pltpu.VMEM((BB,), jnp.int32),
            pltpu.VMEM((BB,), jnp.int32),
            pltpu.VMEM((BB, DW), jnp.float32),
            pltpu.VMEM((BB * H,), jnp.float32),
            pltpu.VMEM((BB, OUT), jnp.float32),
            pltpu.VMEM_SHARED((NPAD, OUT), jnp.float32),
            pltpu.SemaphoreType.DMA,
        ],
    )


def _cnt_body(src_hbm, dst_hbm, z_hbm, cnt_out,
              src_v, dst_v, wbuf_v, msg_v, cnt_sp):
    cid = lax.axis_index("c")
    sid = lax.axis_index("s")
    wid = sid * NC + cid
    row0 = sid * ROWS_PER_SID
    pltpu.sync_copy(z_hbm, cnt_sp.at[pl.ds(row0, ROWS_PER_SID)])
    plsc.subcore_barrier()
    base0 = wid * EPW

    def chunk(i, carry):
        base = pl.multiple_of(base0 + i * BB, 8)
        pltpu.sync_copy(src_hbm.at[pl.ds(base, BB)], src_v)
        pltpu.sync_copy(dst_hbm.at[pl.ds(base, BB)], dst_v)
        sg = src_v[pl.ds(0, 16)]
        dg = dst_v[pl.ds(0, 16)]
        wbuf_v[pl.ds(0, 16)] = jnp.where(sg != dg, 1.0, 0.0)

        def edge(e, c2):
            wv = plsc.load_gather(wbuf_v, [jnp.full((16,), 0, jnp.int32) + e])
            for k in range(OUT // 16):
                msg_v[e, pl.ds(k * 16, 16)] = wv
            return c2

        lax.fori_loop(0, BB, edge, 0)
        pltpu.sync_copy(msg_v, cnt_sp.at[dst_v], add=True)
        return carry

    lax.fori_loop(0, CB, chunk, 0)

    plsc.subcore_barrier()
    pltpu.sync_copy(cnt_sp.at[pl.ds(row0, ROWS_PER_SID)],
                    cnt_out.at[cid, pl.ds(row0, ROWS_PER_SID)])


def _make_cnt_kernel():
    return pl.kernel(
        _cnt_body,
        out_type=[jax.ShapeDtypeStruct((NC, NPAD, OUT), jnp.float32)],
        mesh=_sc_mesh(),
        compiler_params=_SC_PARAMS,
        scratch_types=[
            pltpu.VMEM((BB,), jnp.int32),
            pltpu.VMEM((BB,), jnp.int32),
            pltpu.VMEM((BB,), jnp.float32),
            pltpu.VMEM((BB, OUT), jnp.float32),
            pltpu.VMEM_SHARED((NPAD, OUT), jnp.float32),
        ],
    )


_sc_kernel_cache = {}


def _get_sc_kernel(kind):
    if kind not in _sc_kernel_cache:
        maker = {"coef": _make_coef_kernel, "agg": _make_agg_kernel,
                 "cnt": _make_cnt_kernel}[kind]
        _sc_kernel_cache[kind] = maker()
    return _sc_kernel_cache[kind]


def _node_kernel(x_ref, scale_ref, shift_ref, wt_ref, u_ref, c_ref,
                 y_ref, yut_ref, self_ref):
    xn = x_ref[...] * scale_ref[...] + shift_ref[...]
    y = jnp.dot(xn, wt_ref[...], preferred_element_type=jnp.float32)
    y_ref[...] = y
    yut_ref[...] = jnp.dot(xn, u_ref[...], preferred_element_type=jnp.float32)
    c = c_ref[...]
    p = jax.nn.softmax(c, axis=1)
    self_ref[...] = jnp.sum(
        y.reshape(NBLK, H, OUT) * p.reshape(1, H, 1), axis=1)


def _node_transform(x, scale, shift, wt, u, c):
    return pl.pallas_call(
        _node_kernel,
        grid=(GRID,),
        in_specs=[
            pl.BlockSpec((NBLK, D), lambda i: (i, 0)),
            pl.BlockSpec((1, D), lambda i: (0, 0)),
            pl.BlockSpec((1, D), lambda i: (0, 0)),
            pl.BlockSpec((D, DW), lambda i: (0, 0)),
            pl.BlockSpec((D, H), lambda i: (0, 0)),
            pl.BlockSpec((1, H), lambda i: (0, 0)),
        ],
        out_specs=[
            pl.BlockSpec((NBLK, DW), lambda i: (i, 0)),
            pl.BlockSpec((NBLK, H), lambda i: (i, 0)),
            pl.BlockSpec((NBLK, OUT), lambda i: (i, 0)),
        ],
        out_shape=[
            jax.ShapeDtypeStruct((N, DW), jnp.float32),
            jax.ShapeDtypeStruct((N, H), jnp.float32),
            jax.ShapeDtypeStruct((N, OUT), jnp.float32),
        ],
    )(x, scale, shift, wt, u, c)


def _post_kernel_mid(agg_ref, cnt_ref, self_ref, b_ref,
                     out_ref, sum_ref, sq_ref):
    a = agg_ref[0, :, :OUT] + agg_ref[1, :, :OUT] + self_ref[...]
    t = a / cnt_ref[...] + b_ref[...]
    t = jnp.maximum(t, 0.0)
    out_ref[...] = t
    i = pl.program_id(0)
    sum_ref[pl.ds(i, 1), :] = jnp.sum(t, axis=0, keepdims=True)
    sq_ref[pl.ds(i, 1), :] = jnp.sum(t * t, axis=0, keepdims=True)


def _post_kernel_last(agg_ref, cnt_ref, self_ref, b_ref, out_ref):
    a = agg_ref[0, :, :OUT] + agg_ref[1, :, :OUT] + self_ref[...]
    out_ref[...] = a / cnt_ref[...] + b_ref[...]


def _post(aggp, cnt, selfmsg, bias, last):
    outw = aggp.shape[-1]
    in_specs = [
        pl.BlockSpec((NC, NBLK, outw), lambda i: (0, i, 0)),
        pl.BlockSpec((NBLK, 1), lambda i: (i, 0)),
        pl.BlockSpec((NBLK, OUT), lambda i: (i, 0)),
        pl.BlockSpec((1, OUT), lambda i: (0, 0)),
    ]
    if last:
        return pl.pallas_call(
            _post_kernel_last,
            grid=(GRID,),
            in_specs=in_specs,
            out_specs=pl.BlockSpec((NBLK, OUT), lambda i: (i, 0)),
            out_shape=jax.ShapeDtypeStruct((N, OUT), jnp.float32),
        )(aggp, cnt, selfmsg, bias)
    return pl.pallas_call(
        _post_kernel_mid,
        grid=(GRID,),
        in_specs=in_specs,
        out_specs=[
            pl.BlockSpec((NBLK, OUT), lambda i: (i, 0)),
            pl.BlockSpec((8, OUT), lambda i: (0, 0)),
            pl.BlockSpec((8, OUT), lambda i: (0, 0)),
        ],
        out_shape=[
            jax.ShapeDtypeStruct((N, OUT), jnp.float32),
            jax.ShapeDtypeStruct((8, OUT), jnp.float32),
            jax.ShapeDtypeStruct((8, OUT), jnp.float32),
        ],
    )(aggp, cnt, selfmsg, bias)


def kernel(x, edge_index, W0, U0, C0, B0, W1, U1, C1, B1, W2, U2, C2, B2,
           G0, BT0, G1, BT1):
    src = edge_index[0]
    dst = edge_index[1]

    ones = jnp.ones((1, D), jnp.float32)
    zeros = jnp.zeros((1, D), jnp.float32)

    z = jnp.zeros((ROWS_PER_SID, OUT), jnp.float32)
    (cntp,) = _get_sc_kernel("cnt")(src, dst, z)
    cnt = (cntp[0, :N, 0] + cntp[1, :N, 0] + 1.0).reshape(N, 1)

    def layer(h, scale, shift, W, U, C, bias, last):
        y, yut, selfmsg = _node_transform(h, scale, shift, W.T, U.T,
                                          C.reshape(1, H))
        (cf,) = _get_sc_kernel("coef")(src, dst, yut.reshape(N * H), C)
        (aggp,) = _get_sc_kernel("agg")(src, dst, y, cf, z)
        return _post(aggp, cnt, selfmsg, bias.reshape(1, OUT), last)

    (t0, s0, q0) = layer(x, ones, zeros, W0, U0, C0, B0, False)
    mu0 = jnp.sum(s0[:GRID], axis=0) / N
    var0 = jnp.sum(q0[:GRID], axis=0) / N - mu0 * mu0
    sc0 = G0 / jnp.sqrt(var0 + EPS)
    sh0 = BT0 - mu0 * sc0

    (t1, s1, q1) = layer(t0, sc0.reshape(1, D), sh0.reshape(1, D),
                         W1, U1, C1, B1, False)
    mu1 = jnp.sum(s1[:GRID], axis=0) / N
    var1 = jnp.sum(q1[:GRID], axis=0) / N - mu1 * mu1
    sc1 = G1 / jnp.sqrt(var1 + EPS)
    sh1 = BT1 - mu1 * sc1

    out = layer(t1, sc1.reshape(1, D), sh1.reshape(1, D),
                W2, U2, C2, B2, True)
    return out


# speedup vs baseline: 3.3434x; 1.4172x over previous
import functools

import jax
import jax.numpy as jnp
from jax import lax
from jax.experimental import pallas as pl
from jax.experimental.pallas import tpu as pltpu
from jax.experimental.pallas import tpu_sc as plsc

N = 10000
E = 320000
D = 128
H = 8
OUT = 128
DW = H * OUT
EPS = 1e-5

NC = 2
NS = 16
NW = NC * NS
EPW = E // NW
BA = 80
CA = EPW // BA
BB = 16
CB = EPW // BB
NPAD = 10240
ROWS_PER_SID = NPAD // NS

NBLK = 2000
GRID = N // NBLK

_SC_PARAMS = pltpu.CompilerParams(needs_layout_passes=False)


def _sc_mesh():
    return plsc.VectorSubcoreMesh(core_axis_name="c", subcore_axis_name="s",
                                  num_cores=NC, num_subcores=NS)


def _coef_body(src_hbm, dst_hbm, yut_hbm, c_hbm, cf_out,
               src_v, dst_v, tab_v, cbuf_v, coef_v):
    cid = lax.axis_index("c")
    sid = lax.axis_index("s")
    wid = sid * NC + cid
    pltpu.sync_copy(yut_hbm, tab_v)
    pltpu.sync_copy(c_hbm, cbuf_v)
    iota = lax.iota(jnp.int32, 16)
    ch = [plsc.load_gather(cbuf_v, [jnp.full((16,), h, jnp.int32)])
          for h in range(H)]
    base0 = wid * EPW

    def chunk(i, carry):
        base = pl.multiple_of(base0 + i * BA, 8)
        pltpu.sync_copy(src_hbm.at[pl.ds(base, BA)], src_v)
        pltpu.sync_copy(dst_hbm.at[pl.ds(base, BA)], dst_v)
        for g in range(BA // 16):
            sg = src_v[pl.ds(g * 16, 16)]
            dg = dst_v[pl.ds(g * 16, 16)]
            w = jnp.where(sg != dg, 1.0, 0.0)
            sg8 = sg * H
            dg8 = dg * H
            dif = []
            for h in range(H):
                hv = jnp.full((16,), h, jnp.int32)
                a = plsc.load_gather(tab_v, [dg8 + hv]) + ch[h]
                b = plsc.load_gather(tab_v, [sg8 + hv])
                dif.append(a - b)
            m = dif[0]
            for h in range(1, H):
                m = jnp.maximum(m, dif[h])
            ex = [jnp.exp(dif[h] - m) for h in range(H)]
            tot = ex[0]
            for h in range(1, H):
                tot = tot + ex[h]
            scale = w / tot
            rows = (g * 16 + iota) * H
            for h in range(H):
                plsc.store_scatter(coef_v, [rows + h], ex[h] * scale)
        pltpu.sync_copy(coef_v, cf_out.at[pl.ds(base * H, BA * H)])
        return carry

    lax.fori_loop(0, CA, chunk, 0)


def _make_coef_kernel():
    return pl.kernel(
        _coef_body,
        out_type=[jax.ShapeDtypeStruct((E * H,), jnp.float32)],
        mesh=_sc_mesh(),
        compiler_params=_SC_PARAMS,
        scratch_types=[
            pltpu.VMEM((BA,), jnp.int32),
            pltpu.VMEM((BA,), jnp.int32),
            pltpu.VMEM((H * N,), jnp.float32),
            pltpu.VMEM((H,), jnp.float32),
            pltpu.VMEM((BA * H,), jnp.float32),
        ],
    )


def _agg_body(src_hbm, dst_hbm, y_hbm, cf_hbm, z_hbm, agg_out,
              src_v0, src_v1, dst_v0, dst_v1, y_v0, y_v1, cf_v0, cf_v1,
              msg_v, agg_sp, gsem0, gsem1):
    cid = lax.axis_index("c")
    sid = lax.axis_index("s")
    wid = sid * NC + cid
    row0 = sid * ROWS_PER_SID
    pltpu.sync_copy(z_hbm, agg_sp.at[pl.ds(row0, ROWS_PER_SID)])
    plsc.subcore_barrier()
    base0 = wid * EPW

    src_b = (src_v0, src_v1)
    dst_b = (dst_v0, dst_v1)
    y_b = (y_v0, y_v1)
    cf_b = (cf_v0, cf_v1)
    gsem_b = (gsem0, gsem1)

    def load_idx(i, b):
        base = pl.multiple_of(base0 + i * BB, 8)
        pltpu.sync_copy(src_hbm.at[pl.ds(base, BB)], src_b[b])
        pltpu.sync_copy(dst_hbm.at[pl.ds(base, BB)], dst_b[b])
        pltpu.sync_copy(cf_hbm.at[pl.ds(base * H, BB * H)], cf_b[b])

    def start_gather(b):
        pltpu.async_copy(y_hbm.at[src_b[b]], y_b[b], gsem_b[b])

    def compute(b):
        y_v = y_b[b]
        cf_v = cf_b[b]

        def edge(e, c2):
            accs = [jnp.zeros((16,), jnp.float32) for _ in range(OUT // 16)]
            for h in range(H):
                ah = plsc.load_gather(
                    cf_v, [jnp.full((16,), h, jnp.int32) + e * H])
                for k in range(OUT // 16):
                    accs[k] = accs[k] + ah * y_v[e, pl.ds(h * OUT + k * 16, 16)]
            for k in range(OUT // 16):
                msg_v[e, pl.ds(k * 16, 16)] = accs[k]
            return c2

        lax.fori_loop(0, BB, edge, 0)
        pltpu.sync_copy(msg_v, agg_sp.at[dst_b[b]], add=True)

    load_idx(0, 0)
    start_gather(0)

    def two_chunks(i, carry):
        load_idx(i + 1, 1)
        start_gather(1)
        pltpu.make_async_copy(y_hbm.at[src_b[0]], y_b[0], gsem_b[0]).wait()
        compute(0)
        load_idx(i + 2, 0)
        start_gather(0)
        pltpu.make_async_copy(y_hbm.at[src_b[1]], y_b[1], gsem_b[1]).wait()
        compute(1)
        return carry

    lax.fori_loop(0, CB // 2, lambda j, c: two_chunks(j * 2, c), 0)
    pltpu.make_async_copy(y_hbm.at[src_b[0]], y_b[0], gsem_b[0]).wait()
    compute(0)

    plsc.subcore_barrier()
    pltpu.sync_copy(agg_sp.at[pl.ds(row0, ROWS_PER_SID)],
                    agg_out.at[cid, pl.ds(row0, ROWS_PER_SID)])


def _make_agg_kernel():
    return pl.kernel(
        _agg_body,
        out_type=[jax.ShapeDtypeStruct((NC, NPAD, OUT), jnp.float32)],
        mesh=_sc_mesh(),
        compiler_params=_SC_PARAMS,
        scratch_types=[
            pltpu.VMEM((BB,), jnp.int32),
            pltpu.VMEM((BB,), jnp.int32),
            pltpu.VMEM((BB,), jnp.int32),
            pltpu.VMEM((BB,), jnp.int32),
            pltpu.VMEM((BB, DW), jnp.float32),
            pltpu.VMEM((BB, DW), jnp.float32),
            pltpu.VMEM((BB * H,), jnp.float32),
            pltpu.VMEM((BB * H,), jnp.float32),
            pltpu.VMEM((BB, OUT), jnp.float32),
            pltpu.VMEM_SHARED((NPAD, OUT), jnp.float32),
            pltpu.SemaphoreType.DMA,
            pltpu.SemaphoreType.DMA,
        ],
    )


def _cnt_body(src_hbm, dst_hbm, z_hbm, cnt_out,
              src_v, dst_v, wbuf_v, msg_v, cnt_sp):
    cid = lax.axis_index("c")
    sid = lax.axis_index("s")
    wid = sid * NC + cid
    row0 = sid * ROWS_PER_SID
    pltpu.sync_copy(z_hbm, cnt_sp.at[pl.ds(row0, ROWS_PER_SID)])
    plsc.subcore_barrier()
    base0 = wid * EPW

    def chunk(i, carry):
        base = pl.multiple_of(base0 + i * BB, 8)
        pltpu.sync_copy(src_hbm.at[pl.ds(base, BB)], src_v)
        pltpu.sync_copy(dst_hbm.at[pl.ds(base, BB)], dst_v)
        sg = src_v[pl.ds(0, 16)]
        dg = dst_v[pl.ds(0, 16)]
        wbuf_v[pl.ds(0, 16)] = jnp.where(sg != dg, 1.0, 0.0)

        def edge(e, c2):
            wv = plsc.load_gather(wbuf_v, [jnp.full((16,), 0, jnp.int32) + e])
            for k in range(OUT // 16):
                msg_v[e, pl.ds(k * 16, 16)] = wv
            return c2

        lax.fori_loop(0, BB, edge, 0)
        pltpu.sync_copy(msg_v, cnt_sp.at[dst_v], add=True)
        return carry

    lax.fori_loop(0, CB, chunk, 0)

    plsc.subcore_barrier()
    pltpu.sync_copy(cnt_sp.at[pl.ds(row0, ROWS_PER_SID)],
                    cnt_out.at[cid, pl.ds(row0, ROWS_PER_SID)])


def _make_cnt_kernel():
    return pl.kernel(
        _cnt_body,
        out_type=[jax.ShapeDtypeStruct((NC, NPAD, OUT), jnp.float32)],
        mesh=_sc_mesh(),
        compiler_params=_SC_PARAMS,
        scratch_types=[
            pltpu.VMEM((BB,), jnp.int32),
            pltpu.VMEM((BB,), jnp.int32),
            pltpu.VMEM((BB,), jnp.float32),
            pltpu.VMEM((BB, OUT), jnp.float32),
            pltpu.VMEM_SHARED((NPAD, OUT), jnp.float32),
        ],
    )


_sc_kernel_cache = {}


def _get_sc_kernel(kind):
    if kind not in _sc_kernel_cache:
        maker = {"coef": _make_coef_kernel, "agg": _make_agg_kernel,
                 "cnt": _make_cnt_kernel}[kind]
        _sc_kernel_cache[kind] = maker()
    return _sc_kernel_cache[kind]


def _node_kernel(x_ref, scale_ref, shift_ref, wt_ref, u_ref, c_ref,
                 y_ref, yut_ref, self_ref):
    xn = x_ref[...] * scale_ref[...] + shift_ref[...]
    y = jnp.dot(xn, wt_ref[...], preferred_element_type=jnp.float32)
    y_ref[...] = y
    yut_ref[...] = jnp.dot(xn, u_ref[...], preferred_element_type=jnp.float32)
    c = c_ref[...]
    p = jax.nn.softmax(c, axis=1)
    self_ref[...] = jnp.sum(
        y.reshape(NBLK, H, OUT) * p.reshape(1, H, 1), axis=1)


def _node_transform(x, scale, shift, wt, u, c):
    return pl.pallas_call(
        _node_kernel,
        grid=(GRID,),
        in_specs=[
            pl.BlockSpec((NBLK, D), lambda i: (i, 0)),
            pl.BlockSpec((1, D), lambda i: (0, 0)),
            pl.BlockSpec((1, D), lambda i: (0, 0)),
            pl.BlockSpec((D, DW), lambda i: (0, 0)),
            pl.BlockSpec((D, H), lambda i: (0, 0)),
            pl.BlockSpec((1, H), lambda i: (0, 0)),
        ],
        out_specs=[
            pl.BlockSpec((NBLK, DW), lambda i: (i, 0)),
            pl.BlockSpec((NBLK, H), lambda i: (i, 0)),
            pl.BlockSpec((NBLK, OUT), lambda i: (i, 0)),
        ],
        out_shape=[
            jax.ShapeDtypeStruct((N, DW), jnp.float32),
            jax.ShapeDtypeStruct((N, H), jnp.float32),
            jax.ShapeDtypeStruct((N, OUT), jnp.float32),
        ],
    )(x, scale, shift, wt, u, c)


def _post_kernel_mid(agg_ref, cnt_ref, self_ref, b_ref,
                     out_ref, sum_ref, sq_ref):
    a = agg_ref[0, :, :OUT] + agg_ref[1, :, :OUT] + self_ref[...]
    t = a / cnt_ref[...] + b_ref[...]
    t = jnp.maximum(t, 0.0)
    out_ref[...] = t
    i = pl.program_id(0)
    sum_ref[pl.ds(i, 1), :] = jnp.sum(t, axis=0, keepdims=True)
    sq_ref[pl.ds(i, 1), :] = jnp.sum(t * t, axis=0, keepdims=True)


def _post_kernel_last(agg_ref, cnt_ref, self_ref, b_ref, out_ref):
    a = agg_ref[0, :, :OUT] + agg_ref[1, :, :OUT] + self_ref[...]
    out_ref[...] = a / cnt_ref[...] + b_ref[...]


def _post(aggp, cnt, selfmsg, bias, last):
    outw = aggp.shape[-1]
    in_specs = [
        pl.BlockSpec((NC, NBLK, outw), lambda i: (0, i, 0)),
        pl.BlockSpec((NBLK, 1), lambda i: (i, 0)),
        pl.BlockSpec((NBLK, OUT), lambda i: (i, 0)),
        pl.BlockSpec((1, OUT), lambda i: (0, 0)),
    ]
    if last:
        return pl.pallas_call(
            _post_kernel_last,
            grid=(GRID,),
            in_specs=in_specs,
            out_specs=pl.BlockSpec((NBLK, OUT), lambda i: (i, 0)),
            out_shape=jax.ShapeDtypeStruct((N, OUT), jnp.float32),
        )(aggp, cnt, selfmsg, bias)
    return pl.pallas_call(
        _post_kernel_mid,
        grid=(GRID,),
        in_specs=in_specs,
        out_specs=[
            pl.BlockSpec((NBLK, OUT), lambda i: (i, 0)),
            pl.BlockSpec((8, OUT), lambda i: (0, 0)),
            pl.BlockSpec((8, OUT), lambda i: (0, 0)),
        ],
        out_shape=[
            jax.ShapeDtypeStruct((N, OUT), jnp.float32),
            jax.ShapeDtypeStruct((8, OUT), jnp.float32),
            jax.ShapeDtypeStruct((8, OUT), jnp.float32),
        ],
    )(aggp, cnt, selfmsg, bias)


def kernel(x, edge_index, W0, U0, C0, B0, W1, U1, C1, B1, W2, U2, C2, B2,
           G0, BT0, G1, BT1):
    src = edge_index[0]
    dst = edge_index[1]

    ones = jnp.ones((1, D), jnp.float32)
    zeros = jnp.zeros((1, D), jnp.float32)

    z = jnp.zeros((ROWS_PER_SID, OUT), jnp.float32)
    (cntp,) = _get_sc_kernel("cnt")(src, dst, z)
    cnt = (cntp[0, :N, 0] + cntp[1, :N, 0] + 1.0).reshape(N, 1)

    def layer(h, scale, shift, W, U, C, bias, last):
        y, yut, selfmsg = _node_transform(h, scale, shift, W.T, U.T,
                                          C.reshape(1, H))
        (cf,) = _get_sc_kernel("coef")(src, dst, yut.reshape(N * H), C)
        (aggp,) = _get_sc_kernel("agg")(src, dst, y, cf, z)
        return _post(aggp, cnt, selfmsg, bias.reshape(1, OUT), last)

    (t0, s0, q0) = layer(x, ones, zeros, W0, U0, C0, B0, False)
    mu0 = jnp.sum(s0[:GRID], axis=0) / N
    var0 = jnp.sum(q0[:GRID], axis=0) / N - mu0 * mu0
    sc0 = G0 / jnp.sqrt(var0 + EPS)
    sh0 = BT0 - mu0 * sc0

    (t1, s1, q1) = layer(t0, sc0.reshape(1, D), sh0.reshape(1, D),
                         W1, U1, C1, B1, False)
    mu1 = jnp.sum(s1[:GRID], axis=0) / N
    var1 = jnp.sum(q1[:GRID], axis=0) / N - mu1 * mu1
    sc1 = G1 / jnp.sqrt(var1 + EPS)
    sh1 = BT1 - mu1 * sc1

    out = layer(t1, sc1.reshape(1, D), sh1.reshape(1, D),
                W2, U2, C2, B2, True)
    return out


# speedup vs baseline: 3.8170x; 1.1417x over previous
import functools

import jax
import jax.numpy as jnp
from jax import lax
from jax.experimental import pallas as pl
from jax.experimental.pallas import tpu as pltpu
from jax.experimental.pallas import tpu_sc as plsc

N = 10000
E = 320000
D = 128
H = 8
OUT = 128
DW = H * OUT
EPS = 1e-5

NC = 2
NS = 16
NW = NC * NS
EPW = E // NW
BA = 80
CA = EPW // BA
BB = 16
CB = EPW // BB
NPAD = 10240
ROWS_PER_SID = NPAD // NS

NBLK = 2000
GRID = N // NBLK

_SC_PARAMS = pltpu.CompilerParams(needs_layout_passes=False)


def _sc_mesh():
    return plsc.VectorSubcoreMesh(core_axis_name="c", subcore_axis_name="s",
                                  num_cores=NC, num_subcores=NS)


def _coef_body(src_hbm, dst_hbm, yut_hbm, c_hbm, cf_out,
               src_v, dst_v, tab_v, cbuf_v, coef_v):
    cid = lax.axis_index("c")
    sid = lax.axis_index("s")
    wid = sid * NC + cid
    pltpu.sync_copy(yut_hbm, tab_v)
    pltpu.sync_copy(c_hbm, cbuf_v)
    iota = lax.iota(jnp.int32, 16)
    ch = [plsc.load_gather(cbuf_v, [jnp.full((16,), h, jnp.int32)])
          for h in range(H)]
    base0 = wid * EPW

    def chunk(i, carry):
        base = pl.multiple_of(base0 + i * BA, 8)
        pltpu.sync_copy(src_hbm.at[pl.ds(base, BA)], src_v)
        pltpu.sync_copy(dst_hbm.at[pl.ds(base, BA)], dst_v)
        for g in range(BA // 16):
            sg = src_v[pl.ds(g * 16, 16)]
            dg = dst_v[pl.ds(g * 16, 16)]
            w = jnp.where(sg != dg, 1.0, 0.0)
            sg8 = sg * H
            dg8 = dg * H
            dif = []
            for h in range(H):
                hv = jnp.full((16,), h, jnp.int32)
                a = plsc.load_gather(tab_v, [dg8 + hv]) + ch[h]
                b = plsc.load_gather(tab_v, [sg8 + hv])
                dif.append(a - b)
            m = dif[0]
            for h in range(1, H):
                m = jnp.maximum(m, dif[h])
            ex = [jnp.exp(dif[h] - m) for h in range(H)]
            tot = ex[0]
            for h in range(1, H):
                tot = tot + ex[h]
            scale = w / tot
            rows = (g * 16 + iota) * H
            for h in range(H):
                plsc.store_scatter(coef_v, [rows + h], ex[h] * scale)
        pltpu.sync_copy(coef_v, cf_out.at[pl.ds(base * H, BA * H)])
        return carry

    lax.fori_loop(0, CA, chunk, 0)


def _make_coef_kernel():
    return pl.kernel(
        _coef_body,
        out_type=[jax.ShapeDtypeStruct((E * H,), jnp.float32)],
        mesh=_sc_mesh(),
        compiler_params=_SC_PARAMS,
        scratch_types=[
            pltpu.VMEM((BA,), jnp.int32),
            pltpu.VMEM((BA,), jnp.int32),
            pltpu.VMEM((H * N,), jnp.float32),
            pltpu.VMEM((H,), jnp.float32),
            pltpu.VMEM((BA * H,), jnp.float32),
        ],
    )


def _agg_body(src_hbm, dst_hbm, y_hbm, cf_hbm, z_hbm, agg_out,
              src_v0, src_v1, dst_v0, dst_v1, sdst_v0, sdst_v1,
              y_v0, y_v1, cf_v0, cf_v1,
              msg_v0, msg_v1, agg_sp, gsem0, gsem1, ssem0, ssem1):
    cid = lax.axis_index("c")
    sid = lax.axis_index("s")
    wid = sid * NC + cid
    row0 = sid * ROWS_PER_SID
    pltpu.sync_copy(z_hbm, agg_sp.at[pl.ds(row0, ROWS_PER_SID)])
    plsc.subcore_barrier()
    base0 = wid * EPW

    src_b = (src_v0, src_v1)
    dst_b = (dst_v0, dst_v1)
    sdst_b = (sdst_v0, sdst_v1)
    y_b = (y_v0, y_v1)
    cf_b = (cf_v0, cf_v1)
    msg_b = (msg_v0, msg_v1)
    gsem_b = (gsem0, gsem1)
    ssem_b = (ssem0, ssem1)

    def load_idx(i, b):
        base = pl.multiple_of(base0 + i * BB, 8)
        pltpu.sync_copy(src_hbm.at[pl.ds(base, BB)], src_b[b])
        pltpu.sync_copy(dst_hbm.at[pl.ds(base, BB)], dst_b[b])
        pltpu.sync_copy(cf_hbm.at[pl.ds(base * H, BB * H)], cf_b[b])

    def start_gather(b):
        pltpu.async_copy(y_hbm.at[src_b[b]], y_b[b], gsem_b[b])

    def compute(b):
        y_v = y_b[b]
        cf_v = cf_b[b]
        msg_v = msg_b[b]
        pltpu.make_async_copy(msg_v, agg_sp.at[sdst_b[b]], ssem_b[b]).wait()

        def edge(e, c2):
            accs = [jnp.zeros((16,), jnp.float32) for _ in range(OUT // 16)]
            for h in range(H):
                ah = plsc.load_gather(
                    cf_v, [jnp.full((16,), h, jnp.int32) + e * H])
                for k in range(OUT // 16):
                    accs[k] = accs[k] + ah * y_v[e, pl.ds(h * OUT + k * 16, 16)]
            for k in range(OUT // 16):
                msg_v[e, pl.ds(k * 16, 16)] = accs[k]
            return c2

        lax.fori_loop(0, BB, edge, 0)
        sdst_b[b][pl.ds(0, 16)] = dst_b[b][pl.ds(0, 16)]
        pltpu.async_copy(msg_v, agg_sp.at[sdst_b[b]], ssem_b[b], add=True)

    load_idx(0, 0)
    start_gather(0)
    for b in range(2):
        pltpu.sync_copy(z_hbm.at[pl.ds(0, BB)], msg_b[b])
        sdst_b[b][pl.ds(0, 16)] = src_b[0][pl.ds(0, 16)]
        pltpu.async_copy(msg_b[b], agg_sp.at[sdst_b[b]], ssem_b[b], add=True)

    def two_chunks(i, carry):
        load_idx(i + 1, 1)
        start_gather(1)
        pltpu.make_async_copy(y_hbm.at[src_b[0]], y_b[0], gsem_b[0]).wait()
        compute(0)
        load_idx(i + 2, 0)
        start_gather(0)
        pltpu.make_async_copy(y_hbm.at[src_b[1]], y_b[1], gsem_b[1]).wait()
        compute(1)
        return carry

    lax.fori_loop(0, CB // 2, lambda j, c: two_chunks(j * 2, c), 0)
    pltpu.make_async_copy(y_hbm.at[src_b[0]], y_b[0], gsem_b[0]).wait()
    compute(0)
    pltpu.make_async_copy(msg_b[0], agg_sp.at[sdst_b[0]], ssem_b[0]).wait()
    pltpu.make_async_copy(msg_b[1], agg_sp.at[sdst_b[1]], ssem_b[1]).wait()

    plsc.subcore_barrier()
    pltpu.sync_copy(agg_sp.at[pl.ds(row0, ROWS_PER_SID)],
                    agg_out.at[cid, pl.ds(row0, ROWS_PER_SID)])


def _make_agg_kernel():
    return pl.kernel(
        _agg_body,
        out_type=[jax.ShapeDtypeStruct((NC, NPAD, OUT), jnp.float32)],
        mesh=_sc_mesh(),
        compiler_params=_SC_PARAMS,
        scratch_types=[
            pltpu.VMEM((BB,), jnp.int32),
            pltpu.VMEM((BB,), jnp.int32),
            pltpu.VMEM((BB,), jnp.int32),
            pltpu.VMEM((BB,), jnp.int32),
            pltpu.VMEM((BB,), jnp.int32),
            pltpu.VMEM((BB,), jnp.int32),
            pltpu.VMEM((BB, DW), jnp.float32),
            pltpu.VMEM((BB, DW), jnp.float32),
            pltpu.VMEM((BB * H,), jnp.float32),
            pltpu.VMEM((BB * H,), jnp.float32),
            pltpu.VMEM((BB, OUT), jnp.float32),
            pltpu.VMEM((BB, OUT), jnp.float32),
            pltpu.VMEM_SHARED((NPAD, OUT), jnp.float32),
            pltpu.SemaphoreType.DMA,
            pltpu.SemaphoreType.DMA,
            pltpu.SemaphoreType.DMA,
            pltpu.SemaphoreType.DMA,
        ],
    )


BC = 80
CC = EPW // BC


def _cnt_body(src_hbm, dst_hbm, z_hbm, cnt_out,
              src_v, dst_v, wbuf_v, msg_v, cnt_sp):
    cid = lax.axis_index("c")
    sid = lax.axis_index("s")
    wid = sid * NC + cid
    row0 = sid * ROWS_PER_SID
    pltpu.sync_copy(z_hbm, cnt_sp.at[pl.ds(row0, ROWS_PER_SID)])
    pltpu.sync_copy(z_hbm.at[pl.ds(0, BC)], msg_v)
    plsc.subcore_barrier()
    base0 = wid * EPW

    def chunk(i, carry):
        base = pl.multiple_of(base0 + i * BC, 8)
        pltpu.sync_copy(src_hbm.at[pl.ds(base, BC)], src_v)
        pltpu.sync_copy(dst_hbm.at[pl.ds(base, BC)], dst_v)
        for g in range(BC // 16):
            sg = src_v[pl.ds(g * 16, 16)]
            dg = dst_v[pl.ds(g * 16, 16)]
            wbuf_v[pl.ds(g * 16, 16)] = jnp.where(sg != dg, 1.0, 0.0)

        def edge(e, c2):
            wv = plsc.load_gather(wbuf_v, [jnp.full((16,), 0, jnp.int32) + e])
            msg_v[e, pl.ds(0, 16)] = wv
            return c2

        lax.fori_loop(0, BC, edge, 0)
        pltpu.sync_copy(msg_v, cnt_sp.at[dst_v], add=True)
        return carry

    lax.fori_loop(0, CC, chunk, 0)

    plsc.subcore_barrier()
    pltpu.sync_copy(cnt_sp.at[pl.ds(row0, ROWS_PER_SID)],
                    cnt_out.at[cid, pl.ds(row0, ROWS_PER_SID)])


def _make_cnt_kernel():
    return pl.kernel(
        _cnt_body,
        out_type=[jax.ShapeDtypeStruct((NC, NPAD, OUT), jnp.float32)],
        mesh=_sc_mesh(),
        compiler_params=_SC_PARAMS,
        scratch_types=[
            pltpu.VMEM((BC,), jnp.int32),
            pltpu.VMEM((BC,), jnp.int32),
            pltpu.VMEM((BC,), jnp.float32),
            pltpu.VMEM((BC, OUT), jnp.float32),
            pltpu.VMEM_SHARED((NPAD, OUT), jnp.float32),
        ],
    )


_sc_kernel_cache = {}


def _get_sc_kernel(kind):
    if kind not in _sc_kernel_cache:
        maker = {"coef": _make_coef_kernel, "agg": _make_agg_kernel,
                 "cnt": _make_cnt_kernel}[kind]
        _sc_kernel_cache[kind] = maker()
    return _sc_kernel_cache[kind]


def _node_kernel(x_ref, scale_ref, shift_ref, wt_ref, u_ref, c_ref,
                 y_ref, yut_ref, self_ref):
    xn = x_ref[...] * scale_ref[...] + shift_ref[...]
    y = jnp.dot(xn, wt_ref[...], preferred_element_type=jnp.float32)
    y_ref[...] = y
    yut_ref[...] = jnp.dot(xn, u_ref[...], preferred_element_type=jnp.float32)
    c = c_ref[...]
    p = jax.nn.softmax(c, axis=1)
    self_ref[...] = jnp.sum(
        y.reshape(NBLK, H, OUT) * p.reshape(1, H, 1), axis=1)


def _node_transform(x, scale, shift, wt, u, c):
    return pl.pallas_call(
        _node_kernel,
        grid=(GRID,),
        in_specs=[
            pl.BlockSpec((NBLK, D), lambda i: (i, 0)),
            pl.BlockSpec((1, D), lambda i: (0, 0)),
            pl.BlockSpec((1, D), lambda i: (0, 0)),
            pl.BlockSpec((D, DW), lambda i: (0, 0)),
            pl.BlockSpec((D, H), lambda i: (0, 0)),
            pl.BlockSpec((1, H), lambda i: (0, 0)),
        ],
        out_specs=[
            pl.BlockSpec((NBLK, DW), lambda i: (i, 0)),
            pl.BlockSpec((NBLK, H), lambda i: (i, 0)),
            pl.BlockSpec((NBLK, OUT), lambda i: (i, 0)),
        ],
        out_shape=[
            jax.ShapeDtypeStruct((N, DW), jnp.float32),
            jax.ShapeDtypeStruct((N, H), jnp.float32),
            jax.ShapeDtypeStruct((N, OUT), jnp.float32),
        ],
    )(x, scale, shift, wt, u, c)


def _post_kernel_mid(agg_ref, cnt_ref, self_ref, b_ref,
                     out_ref, sum_ref, sq_ref):
    a = agg_ref[0, :, :OUT] + agg_ref[1, :, :OUT] + self_ref[...]
    t = a / cnt_ref[...] + b_ref[...]
    t = jnp.maximum(t, 0.0)
    out_ref[...] = t
    i = pl.program_id(0)
    sum_ref[pl.ds(i, 1), :] = jnp.sum(t, axis=0, keepdims=True)
    sq_ref[pl.ds(i, 1), :] = jnp.sum(t * t, axis=0, keepdims=True)


def _post_kernel_last(agg_ref, cnt_ref, self_ref, b_ref, out_ref):
    a = agg_ref[0, :, :OUT] + agg_ref[1, :, :OUT] + self_ref[...]
    out_ref[...] = a / cnt_ref[...] + b_ref[...]


def _post(aggp, cnt, selfmsg, bias, last):
    outw = aggp.shape[-1]
    in_specs = [
        pl.BlockSpec((NC, NBLK, outw), lambda i: (0, i, 0)),
        pl.BlockSpec((NBLK, 1), lambda i: (i, 0)),
        pl.BlockSpec((NBLK, OUT), lambda i: (i, 0)),
        pl.BlockSpec((1, OUT), lambda i: (0, 0)),
    ]
    if last:
        return pl.pallas_call(
            _post_kernel_last,
            grid=(GRID,),
            in_specs=in_specs,
            out_specs=pl.BlockSpec((NBLK, OUT), lambda i: (i, 0)),
            out_shape=jax.ShapeDtypeStruct((N, OUT), jnp.float32),
        )(aggp, cnt, selfmsg, bias)
    return pl.pallas_call(
        _post_kernel_mid,
        grid=(GRID,),
        in_specs=in_specs,
        out_specs=[
            pl.BlockSpec((NBLK, OUT), lambda i: (i, 0)),
            pl.BlockSpec((8, OUT), lambda i: (0, 0)),
            pl.BlockSpec((8, OUT), lambda i: (0, 0)),
        ],
        out_shape=[
            jax.ShapeDtypeStruct((N, OUT), jnp.float32),
            jax.ShapeDtypeStruct((8, OUT), jnp.float32),
            jax.ShapeDtypeStruct((8, OUT), jnp.float32),
        ],
    )(aggp, cnt, selfmsg, bias)


def kernel(x, edge_index, W0, U0, C0, B0, W1, U1, C1, B1, W2, U2, C2, B2,
           G0, BT0, G1, BT1):
    src = edge_index[0]
    dst = edge_index[1]

    ones = jnp.ones((1, D), jnp.float32)
    zeros = jnp.zeros((1, D), jnp.float32)

    z = jnp.zeros((ROWS_PER_SID, OUT), jnp.float32)
    (cntp,) = _get_sc_kernel("cnt")(src, dst, z)
    cnt = (cntp[0, :N, 0] + cntp[1, :N, 0] + 1.0).reshape(N, 1)

    def layer(h, scale, shift, W, U, C, bias, last):
        y, yut, selfmsg = _node_transform(h, scale, shift, W.T, U.T,
                                          C.reshape(1, H))
        (cf,) = _get_sc_kernel("coef")(src, dst, yut.reshape(N * H), C)
        (aggp,) = _get_sc_kernel("agg")(src, dst, y, cf, z)
        return _post(aggp, cnt, selfmsg, bias.reshape(1, OUT), last)

    (t0, s0, q0) = layer(x, ones, zeros, W0, U0, C0, B0, False)
    mu0 = jnp.sum(s0[:GRID], axis=0) / N
    var0 = jnp.sum(q0[:GRID], axis=0) / N - mu0 * mu0
    sc0 = G0 / jnp.sqrt(var0 + EPS)
    sh0 = BT0 - mu0 * sc0

    (t1, s1, q1) = layer(t0, sc0.reshape(1, D), sh0.reshape(1, D),
                         W1, U1, C1, B1, False)
    mu1 = jnp.sum(s1[:GRID], axis=0) / N
    var1 = jnp.sum(q1[:GRID], axis=0) / N - mu1 * mu1
    sc1 = G1 / jnp.sqrt(var1 + EPS)
    sh1 = BT1 - mu1 * sc1

    out = layer(t1, sc1.reshape(1, D), sh1.reshape(1, D),
                W2, U2, C2, B2, True)
    return out
